# Initial kernel scaffold; baseline (speedup 1.0000x reference)
import jax
import jax.numpy as jnp
from jax.experimental import pallas as pl


def kernel(x, edge_index, W1, b1, W2, b2):
    raise NotImplementedError("write your pallas kernel here")


# speedup vs baseline: 200.8052x; 200.8052x over previous
import jax
import jax.numpy as jnp
from jax import lax
from jax.experimental import pallas as pl
from jax.experimental.pallas import tpu as pltpu
from jax.experimental.pallas import tpu_sc as plsc

N_NODES = 100000
N_EDGES = 6400000
NC = 2
NS = 16
NW = NC * NS
L = 16

NP = 100352
NPC = NP // NS
NPW = NP // NW
REAL_LAST_W = N_NODES - 31 * NPW

PW = N_EDGES // NW
CH1 = 5000
CH2 = 2000

_MESH = dict(core_axis_name="c", subcore_axis_name="s")
_PARAMS = dict(
    compiler_params=pltpu.CompilerParams(
        needs_layout_passes=False, use_tc_tiling_on_sc=False
    ),
)


def _mesh():
  return plsc.VectorSubcoreMesh(num_cores=NC, num_subcores=NS, **_MESH)


def _iota():
  return lax.iota(jnp.int32, L)


def _rsqrt_sc(d):
  i = plsc.bitcast(d, jnp.int32)
  i = jnp.int32(0x5F3759DF) - lax.shift_right_logical(i, jnp.int32(1))
  y = plsc.bitcast(i, jnp.float32)
  for _ in range(3):
    y = y * (jnp.float32(1.5) - jnp.float32(0.5) * d * y * y)
  return y


def _worker_id():
  return lax.axis_index("s") * NC + lax.axis_index("c")


def _k1_body(dst_hbm, ones_hbm, zeros1_hbm, cnt_out, deg_sh, didx, ones_v):
  cid = lax.axis_index("c")
  sid = lax.axis_index("s")
  wid = _worker_id()
  n0 = sid * NPC
  pltpu.sync_copy(zeros1_hbm, deg_sh.at[pl.ds(n0, NPC)])
  pltpu.sync_copy(ones_hbm, ones_v)
  plsc.subcore_barrier()
  base = wid * PW

  def step(i, carry):
    pltpu.sync_copy(dst_hbm.at[pl.ds(base + i * CH1, CH1)], didx)
    pltpu.sync_copy(ones_v, deg_sh.at[didx], add=True)
    return carry

  lax.fori_loop(0, PW // CH1, step, 0)
  plsc.subcore_barrier()
  pltpu.sync_copy(deg_sh.at[pl.ds(n0, NPC)], cnt_out.at[cid, pl.ds(n0, NPC)])


def _k1(dst, ones, zeros1):
  f = pl.kernel(
      _k1_body,
      out_type=jax.ShapeDtypeStruct((NC, NP), jnp.float32),
      mesh=_mesh(),
      scratch_types=[
          pltpu.VMEM_SHARED((NP,), jnp.float32),
          pltpu.VMEM((CH1,), jnp.int32),
          pltpu.VMEM((CH1,), jnp.float32),
      ],
      **_PARAMS,
  )
  return f(dst, ones, zeros1)


def _k2_body(cnt_hbm, x_hbm, w1_hbm, dis_out, u1_out,
             x_v, c0_v, c1_v, dis_v, u_buf, wv):
  wid = _worker_id()
  n0 = wid * NPW

  pltpu.sync_copy(w1_hbm, wv)
  pltpu.sync_copy(cnt_hbm.at[0, pl.ds(n0, NPW)], c0_v)
  pltpu.sync_copy(cnt_hbm.at[1, pl.ds(n0, NPW)], c1_v)

  @pl.when(wid < NW - 1)
  def _():
    pltpu.sync_copy(x_hbm.at[pl.ds(n0, NPW)], x_v)

  @pl.when(wid == NW - 1)
  def _():
    pltpu.sync_copy(x_hbm.at[pl.ds(n0, REAL_LAST_W)],
                    x_v.at[pl.ds(0, REAL_LAST_W)])

  wvec0 = wv[pl.ds(0, L)]
  wvec1 = wv[pl.ds(L, L)]
  w1s = [[wvec0[k * 4 + d] if k * 4 + d < L else wvec1[k * 4 + d - L]
          for d in range(4)] for k in range(5)]

  def dense(g, carry):
    r0 = g * L
    it = _iota()
    deg = c0_v[pl.ds(r0, L)] + c1_v[pl.ds(r0, L)] + jnp.float32(1.0)
    dis = _rsqrt_sc(deg)
    dis_v[pl.ds(r0, L)] = dis
    rows_i = r0 + it
    xk = [plsc.load_gather(x_v, [rows_i, jnp.full((L,), k, jnp.int32)])
          for k in range(5)]
    for d in range(4):
      acc = xk[0] * w1s[0][d]
      for k in range(1, 5):
        acc = acc + xk[k] * w1s[k][d]
      plsc.store_scatter(u_buf, [rows_i, jnp.full((L,), d, jnp.int32)],
                         dis * acc)
    return carry

  lax.fori_loop(0, NPW // L, dense, 0)
  pltpu.sync_copy(dis_v, dis_out.at[pl.ds(n0, NPW)])
  pltpu.sync_copy(u_buf, u1_out.at[pl.ds(n0, NPW)])


def _k2(cnt, x, w1p):
  f = pl.kernel(
      _k2_body,
      out_type=(
          jax.ShapeDtypeStruct((NP,), jnp.float32),
          jax.ShapeDtypeStruct((NP, 4), jnp.float32),
      ),
      mesh=_mesh(),
      scratch_types=[
          pltpu.VMEM((NPW, 5), jnp.float32),
          pltpu.VMEM((NPW,), jnp.float32),
          pltpu.VMEM((NPW,), jnp.float32),
          pltpu.VMEM((NPW,), jnp.float32),
          pltpu.VMEM((NPW, 4), jnp.float32),
          pltpu.VMEM((32,), jnp.float32),
      ],
      **_PARAMS,
  )
  return f(cnt, x, w1p)


def _conv_body(ch, u_hbm, src_hbm, dst_hbm, zeros_hbm, acc_out,
               u_sh, acc_sh, sidx, didx, rows):
  cid = lax.axis_index("c")
  sid = lax.axis_index("s")
  wid = _worker_id()
  n0 = sid * NPC
  pltpu.sync_copy(u_hbm.at[pl.ds(n0, NPC)], u_sh.at[pl.ds(n0, NPC)])
  pltpu.sync_copy(zeros_hbm, acc_sh.at[pl.ds(n0, NPC)])
  plsc.subcore_barrier()
  base = wid * PW

  def step(i, carry):
    pltpu.sync_copy(src_hbm.at[pl.ds(base + i * ch, ch)], sidx)
    pltpu.sync_copy(dst_hbm.at[pl.ds(base + i * ch, ch)], didx)
    pltpu.sync_copy(u_sh.at[sidx], rows)
    pltpu.sync_copy(rows, acc_sh.at[didx], add=True)
    return carry

  lax.fori_loop(0, PW // ch, step, 0)
  plsc.subcore_barrier()
  pltpu.sync_copy(acc_sh.at[pl.ds(n0, NPC)], acc_out.at[cid, pl.ds(n0, NPC)])


def _conv(u, src, dst, zeros, d, ch):
  def body(*refs):
    _conv_body(ch, *refs)

  f = pl.kernel(
      body,
      out_type=jax.ShapeDtypeStruct((NC, NP, d), jnp.float32),
      mesh=_mesh(),
      scratch_types=[
          pltpu.VMEM_SHARED((NP, d), jnp.float32),
          pltpu.VMEM_SHARED((NP, d), jnp.float32),
          pltpu.VMEM((ch,), jnp.int32),
          pltpu.VMEM((ch,), jnp.int32),
          pltpu.VMEM((ch, d), jnp.float32),
      ],
      **_PARAMS,
  )
  return f(u, src, dst, zeros)


def _k4_body(acc1_hbm, dis_hbm, u1_hbm, w2_hbm, u2_out,
             a0_v, a1_v, u1_v, dis_v, u2_buf, wv):
  wid = _worker_id()
  n0 = wid * NPW

  pltpu.sync_copy(w2_hbm, wv)
  pltpu.sync_copy(acc1_hbm.at[0, pl.ds(n0, NPW)], a0_v)
  pltpu.sync_copy(acc1_hbm.at[1, pl.ds(n0, NPW)], a1_v)
  pltpu.sync_copy(u1_hbm.at[pl.ds(n0, NPW)], u1_v)
  pltpu.sync_copy(dis_hbm.at[pl.ds(n0, NPW)], dis_v)

  wvecs = [wv[pl.ds(j * L, L)] for j in range(3)]
  w2s = [[wvecs[(d * 8 + c) // L][(d * 8 + c) % L]
          for c in range(8)] for d in range(4)]
  b1s = [wvecs[2][d] for d in range(4)]

  def dense(g, carry):
    r0 = g * L
    it = _iota()
    rows_i = r0 + it
    dis = dis_v[pl.ds(r0, L)]
    hd = []
    for d in range(4):
      cd = jnp.full((L,), d, jnp.int32)
      s = (plsc.load_gather(a0_v, [rows_i, cd])
           + plsc.load_gather(a1_v, [rows_i, cd])
           + plsc.load_gather(u1_v, [rows_i, cd]))
      hd.append(jnp.maximum(dis * s + b1s[d], jnp.float32(0.0)))
    for c in range(8):
      acc = hd[0] * w2s[0][c]
      for d in range(1, 4):
        acc = acc + hd[d] * w2s[d][c]
      plsc.store_scatter(u2_buf, [rows_i, jnp.full((L,), c, jnp.int32)],
                         dis * acc)
    return carry

  lax.fori_loop(0, NPW // L, dense, 0)
  pltpu.sync_copy(u2_buf, u2_out.at[pl.ds(n0, NPW)])


def _k4(acc1, dis, u1, w2p):
  f = pl.kernel(
      _k4_body,
      out_type=jax.ShapeDtypeStruct((NP, 8), jnp.float32),
      mesh=_mesh(),
      scratch_types=[
          pltpu.VMEM((NPW, 4), jnp.float32),
          pltpu.VMEM((NPW, 4), jnp.float32),
          pltpu.VMEM((NPW, 4), jnp.float32),
          pltpu.VMEM((NPW,), jnp.float32),
          pltpu.VMEM((NPW, 8), jnp.float32),
          pltpu.VMEM((48,), jnp.float32),
      ],
      **_PARAMS,
  )
  return f(acc1, dis, u1, w2p)


def _k6_body(acc2_hbm, dis_hbm, u2_hbm, w2_hbm,
             outpre_out, ms_out,
             a0_v, a1_v, u2_v, dis_v, out_buf, wv, msbuf, t16):
  wid = _worker_id()
  n0 = wid * NPW
  nv = lax.select(wid == NW - 1,
                  jnp.int32(REAL_LAST_W * 8 // L),
                  jnp.int32(NPW * 8 // L))

  pltpu.sync_copy(w2_hbm, wv)
  pltpu.sync_copy(acc2_hbm.at[0, pl.ds(n0, NPW)], a0_v)
  pltpu.sync_copy(acc2_hbm.at[1, pl.ds(n0, NPW)], a1_v)
  pltpu.sync_copy(u2_hbm.at[pl.ds(n0, NPW)], u2_v)
  pltpu.sync_copy(dis_hbm.at[pl.ds(n0, NPW)], dis_v)

  it = _iota()
  b2v = plsc.load_gather(wv, [40 + jnp.bitwise_and(it, 7)])

  def pass1(i, m_run):
    rloc = 2 * i + jnp.right_shift(it, 3)
    cloc = jnp.bitwise_and(it, 7)
    val = (plsc.load_gather(a0_v, [rloc, cloc])
           + plsc.load_gather(a1_v, [rloc, cloc])
           + plsc.load_gather(u2_v, [rloc, cloc]))
    val = plsc.load_gather(dis_v, [rloc]) * val + b2v
    plsc.store_scatter(out_buf, [rloc, cloc], val)
    return jnp.maximum(m_run, val)

  m_run = lax.fori_loop(0, nv, pass1, jnp.full((L,), -1e30, jnp.float32))
  t16[pl.ds(0, L)] = m_run
  m2 = jnp.maximum(m_run, plsc.load_gather(t16, [jnp.bitwise_and(it + 8, 15)]))

  def pass2(i, s_run):
    rloc = 2 * i + jnp.right_shift(it, 3)
    cloc = jnp.bitwise_and(it, 7)
    val = plsc.load_gather(out_buf, [rloc, cloc])
    return s_run + jnp.exp(val - m2)

  s_run = lax.fori_loop(0, nv, pass2, jnp.zeros((L,), jnp.float32))
  t16[pl.ds(0, L)] = s_run
  s2 = s_run + plsc.load_gather(t16, [jnp.bitwise_and(it + 8, 15)])

  msbuf[pl.ds(0, L)] = m2
  msbuf[pl.ds(L, L)] = s2
  pltpu.sync_copy(msbuf, ms_out.at[pl.ds(wid * 2 * L, 2 * L)])

  @pl.when(wid < NW - 1)
  def _():
    pltpu.sync_copy(out_buf, outpre_out.at[pl.ds(n0, NPW)])

  @pl.when(wid == NW - 1)
  def _():
    pltpu.sync_copy(out_buf.at[pl.ds(0, REAL_LAST_W)],
                    outpre_out.at[pl.ds(n0, REAL_LAST_W)])


def _k6(acc2, dis, u2, w2p):
  f = pl.kernel(
      _k6_body,
      out_type=(
          jax.ShapeDtypeStruct((NP, 8), jnp.float32),
          jax.ShapeDtypeStruct((NW * 2 * L,), jnp.float32),
      ),
      mesh=_mesh(),
      scratch_types=[
          pltpu.VMEM((NPW, 8), jnp.float32),
          pltpu.VMEM((NPW, 8), jnp.float32),
          pltpu.VMEM((NPW, 8), jnp.float32),
          pltpu.VMEM((NPW,), jnp.float32),
          pltpu.VMEM((NPW, 8), jnp.float32),
          pltpu.VMEM((48,), jnp.float32),
          pltpu.VMEM((2 * L,), jnp.float32),
          pltpu.VMEM((L,), jnp.float32),
      ],
      **_PARAMS,
  )
  return f(acc2, dis, u2, w2p)


def _k7_body(outpre_hbm, ms_hbm, out_hbm, buf, msv):
  wid = _worker_id()
  n0 = wid * NPW
  nrows = lax.select(wid == NW - 1, jnp.int32(REAL_LAST_W), jnp.int32(NPW))
  nv = nrows * 8 // L

  pltpu.sync_copy(ms_hbm, msv)
  it = _iota()

  def combine(w, ms):
    m, s = ms
    mi = msv[pl.ds(w * 2 * L, L)]
    si = msv[pl.ds(w * 2 * L + L, L)]
    mn = jnp.maximum(m, mi)
    s = s * jnp.exp(m - mn) + si * jnp.exp(mi - mn)
    return mn, s

  m, s = lax.fori_loop(0, NW, combine,
                       (jnp.full((L,), -1e30, jnp.float32),
                        jnp.zeros((L,), jnp.float32)))

  bits = plsc.bitcast(s, jnp.int32)
  e = lax.shift_right_logical(bits, jnp.int32(23)) - jnp.int32(127)
  y = e.astype(jnp.float32) * jnp.float32(0.6931472) + jnp.float32(0.3466)
  for _ in range(4):
    y = y + s * jnp.exp(-y) - jnp.float32(1.0)
  lse = m + y

  @pl.when(wid < NW - 1)
  def _():
    pltpu.sync_copy(outpre_hbm.at[pl.ds(n0, NPW)], buf)

  @pl.when(wid == NW - 1)
  def _():
    pltpu.sync_copy(outpre_hbm.at[pl.ds(n0, REAL_LAST_W)],
                    buf.at[pl.ds(0, REAL_LAST_W)])

  def sub(i, carry):
    rloc = 2 * i + jnp.right_shift(it, 3)
    cloc = jnp.bitwise_and(it, 7)
    val = plsc.load_gather(buf, [rloc, cloc]) - lse
    plsc.store_scatter(buf, [rloc, cloc], val)
    return carry

  lax.fori_loop(0, nv, sub, 0)

  @pl.when(wid < NW - 1)
  def _():
    pltpu.sync_copy(buf, out_hbm.at[pl.ds(n0, NPW)])

  @pl.when(wid == NW - 1)
  def _():
    pltpu.sync_copy(buf.at[pl.ds(0, REAL_LAST_W)],
                    out_hbm.at[pl.ds(n0, REAL_LAST_W)])


def _k7(outpre, ms):
  f = pl.kernel(
      _k7_body,
      out_type=jax.ShapeDtypeStruct((N_NODES, 8), jnp.float32),
      mesh=_mesh(),
      scratch_types=[
          pltpu.VMEM((NPW, 8), jnp.float32),
          pltpu.VMEM((NW * 2 * L,), jnp.float32),
      ],
      **_PARAMS,
  )
  return f(outpre, ms)


def kernel(x, edge_index, W1, b1, W2, b2):
  src = edge_index[0].astype(jnp.int32)
  dst = edge_index[1].astype(jnp.int32)
  w1p = jnp.concatenate([W1.reshape(-1).astype(jnp.float32),
                         jnp.zeros((12,), jnp.float32)])
  w2p = jnp.concatenate([W2.reshape(-1).astype(jnp.float32),
                         b1.astype(jnp.float32),
                         jnp.zeros((4,), jnp.float32),
                         b2.astype(jnp.float32)])
  ones = jnp.ones((CH1,), jnp.float32)
  zeros1 = jnp.zeros((NPC,), jnp.float32)
  zeros4 = jnp.zeros((NPC, 4), jnp.float32)
  zeros8 = jnp.zeros((NPC, 8), jnp.float32)

  cnt = _k1(dst, ones, zeros1)
  dis, u1 = _k2(cnt, x, w1p)
  acc1 = _conv(u1, src, dst, zeros4, 4, CH1)
  u2 = _k4(acc1, dis, u1, w2p)
  acc2 = _conv(u2, src, dst, zeros8, 8, CH2)
  outpre, ms = _k6(acc2, dis, u2, w2p)
  return _k7(outpre, ms)
